# Initial kernel scaffold; baseline (speedup 1.0000x reference)
import jax
import jax.numpy as jnp
from jax.experimental import pallas as pl


def kernel(x, edge_index, edge_weight, batch, W1, b1, W2, b2, W3, b3, Wp, bp):
    raise NotImplementedError("write your pallas kernel here")


# speedup vs baseline: 1.9130x; 1.9130x over previous
import functools

import jax
import jax.numpy as jnp
from jax import lax
from jax.experimental import pallas as pl
from jax.experimental.pallas import tpu as pltpu
from jax.experimental.pallas import tpu_sc as plsc

NC = 2
NS = 16
RB = 512
CH = 512
CW = 128

_SC_PARAMS = pltpu.CompilerParams(
    needs_layout_passes=False, use_tc_tiling_on_sc=False
)


def _sc_mesh():
    return plsc.VectorSubcoreMesh(
        core_axis_name="c", subcore_axis_name="s", num_cores=NC, num_subcores=NS
    )


def _make_deg(n2, epad):
    epc = epad // (NC * NS)
    nchunks = epc // CH
    nsl = n2 // NS

    @functools.partial(
        pl.kernel,
        out_type=jax.ShapeDtypeStruct((NC * n2,), jnp.float32),
        mesh=_sc_mesh(),
        scratch_types=[
            pltpu.VMEM_SHARED((n2,), jnp.float32),
            pltpu.VMEM((CH // CW, CW), jnp.int32),
            pltpu.VMEM((CH // CW, CW), jnp.float32),
            pltpu.VMEM((nsl,), jnp.float32),
        ],
        compiler_params=_SC_PARAMS,
    )
    def k(d_idx, w, out, acc, dbuf, wbuf, zbuf):
        c = lax.axis_index("c")
        sid = lax.axis_index("s")
        zv = jnp.zeros((16,), jnp.float32)

        def zfill(i, _):
            zbuf[pl.ds(i * 16, 16)] = zv
            return 0

        lax.fori_loop(0, nsl // 16, zfill, 0)
        pltpu.sync_copy(zbuf, acc.at[pl.ds(sid * nsl, nsl)])
        plsc.subcore_barrier()

        ebase = (c * NS + sid) * epc

        def chunk(ci, _):
            base = ebase + ci * CH
            pltpu.sync_copy(d_idx.at[pl.ds(base // CW, CH // CW)], dbuf)
            pltpu.sync_copy(w.at[pl.ds(base // CW, CH // CW)], wbuf)
            for j in range(CH // CW):
                pltpu.sync_copy(wbuf.at[j], acc.at[dbuf.at[j]], add=True)
            return 0

        lax.fori_loop(0, nchunks, chunk, 0)
        plsc.subcore_barrier()
        lo = sid * nsl
        pltpu.sync_copy(acc.at[pl.ds(lo, nsl)], out.at[pl.ds(c * n2 + lo, nsl)])

    return k


def _make_agg(nb, n2, epad):
    h2 = n2 // NC
    eps_ = epad // NS
    nchunks = eps_ // CH
    nsl = h2 // NS
    mesh = _sc_mesh()

    @functools.partial(
        pl.kernel,
        out_type=jax.ShapeDtypeStruct((nb * n2, 16), jnp.float32),
        mesh=mesh,
        scratch_types=[
            pltpu.VMEM_SHARED((h2, 16), jnp.float32),
            pltpu.VMEM((CH // CW, CW), jnp.int32),
            pltpu.VMEM((CH // CW, CW), jnp.int32),
            pltpu.VMEM((CH // CW, CW), jnp.float32),
            pltpu.VMEM((CH, 16), jnp.float32),
            pltpu.VMEM((nsl, 16), jnp.float32),
            pltpu.SemaphoreType.DMA,
        ],
        compiler_params=_SC_PARAMS,
    )
    def k(hp, s_idx, d_idx, w, out, acc, sbuf, dbuf, wbuf, rows, zrows, sem):
        c = lax.axis_index("c")
        sid = lax.axis_index("s")
        zv = jnp.zeros((16,), jnp.float32)

        def zfill(i, _):
            zrows[i] = zv
            return 0

        lax.fori_loop(0, nsl, zfill, 0)

        ebase = sid * eps_
        dlo = c * h2

        def do_fb(fb):
            pltpu.sync_copy(zrows, acc.at[pl.ds(sid * nsl, nsl)])
            plsc.subcore_barrier()
            off = fb * n2

            def chunk(ci, _):
                base = ebase + ci * CH
                pltpu.sync_copy(s_idx.at[pl.ds(base // CW, CH // CW)], sbuf)
                pltpu.sync_copy(d_idx.at[pl.ds(base // CW, CH // CW)], dbuf)
                pltpu.sync_copy(w.at[pl.ds(base // CW, CH // CW)], wbuf)

                def adj(j, _):
                    for kk in range(CW // 16):
                        sl = pl.ds(kk * 16, 16)
                        dv = dbuf[j, sl]
                        dloc = dv - dlo
                        m = (dloc >= 0) & (dloc < h2)
                        dbuf[j, sl] = jnp.where(m, dloc, 0)
                        wv = wbuf[j, sl]
                        wbuf[j, sl] = jnp.where(m, wv, 0.0)
                        sbuf[j, sl] = sbuf[j, sl] + off
                    return 0

                lax.fori_loop(0, CH // CW, adj, 0)

                cps = []
                for j in range(CH // CW):
                    cps.append(
                        pltpu.async_copy(
                            hp.at[sbuf.at[j]],
                            rows.at[pl.ds(j * CW, CW)],
                            sem,
                        )
                    )
                for cp in cps:
                    cp.wait()

                def scale(j, _):
                    for kk in range(16):
                        i = j * 16 + kk
                        bc = plsc.load_gather(
                            wbuf,
                            [
                                jnp.full((16,), i // CW, jnp.int32),
                                jnp.full((16,), i % CW, jnp.int32),
                            ],
                        )
                        rows[i] = rows[i] * bc
                    return 0

                lax.fori_loop(0, CH // 16, scale, 0)

                for j in range(CH // CW):
                    pltpu.sync_copy(
                        rows.at[pl.ds(j * CW, CW)],
                        acc.at[dbuf.at[j]],
                        add=True,
                    )
                return 0

            lax.fori_loop(0, nchunks, chunk, 0)
            plsc.subcore_barrier()
            lo = sid * nsl
            pltpu.sync_copy(
                acc.at[pl.ds(lo, nsl)],
                out.at[pl.ds(fb * n2 + dlo + lo, nsl)],
            )
            plsc.subcore_barrier()

        for fb in range(nb):
            do_fb(fb)

    return k


def _tc_prep(n2, f1):
    nb = f1 // 16
    ng = n2 // RB

    def body(deg_ref, x_ref, w1_ref, dis_ref, hp_ref):
        deg = deg_ref[0, :] + deg_ref[1, :] + 1.0
        dis = lax.rsqrt(jnp.maximum(deg, 1e-12))
        h = jnp.dot(x_ref[...], w1_ref[...], preferred_element_type=jnp.float32)
        hp = h * dis[:, None]
        dis_ref[...] = dis[:, None]
        for i in range(nb):
            hp_ref[i] = hp[:, i * 16:(i + 1) * 16]

    return pl.pallas_call(
        body,
        grid=(ng,),
        in_specs=[
            pl.BlockSpec((2, RB), lambda r: (0, r)),
            pl.BlockSpec((RB, 32), lambda r: (r, 0)),
            pl.BlockSpec((32, f1), lambda r: (0, 0)),
        ],
        out_specs=[
            pl.BlockSpec((RB, 1), lambda r: (r, 0)),
            pl.BlockSpec((nb, RB, 16), lambda r: (0, r, 0)),
        ],
        out_shape=[
            jax.ShapeDtypeStruct((n2, 1), jnp.float32),
            jax.ShapeDtypeStruct((nb, n2, 16), jnp.float32),
        ],
    )


def _tc_layer(n2, f, f2):
    nb, nb2 = f // 16, f2 // 16
    ng = n2 // RB

    def body(scat_ref, hp_ref, dis_ref, b_ref, w_ref, hp2_ref):
        dis = dis_ref[...]
        parts = [
            dis * (scat_ref[i] + hp_ref[i]) for i in range(nb)
        ]
        m = jnp.concatenate(parts, axis=-1)
        act = jnp.maximum(m + b_ref[...], 0.0)
        h2 = jnp.dot(act, w_ref[...], preferred_element_type=jnp.float32)
        hp2 = h2 * dis
        for i in range(nb2):
            hp2_ref[i] = hp2[:, i * 16:(i + 1) * 16]

    return pl.pallas_call(
        body,
        grid=(ng,),
        in_specs=[
            pl.BlockSpec((nb, RB, 16), lambda r: (0, r, 0)),
            pl.BlockSpec((nb, RB, 16), lambda r: (0, r, 0)),
            pl.BlockSpec((RB, 1), lambda r: (r, 0)),
            pl.BlockSpec((1, f), lambda r: (0, 0)),
            pl.BlockSpec((f, f2), lambda r: (0, 0)),
        ],
        out_specs=pl.BlockSpec((nb2, RB, 16), lambda r: (0, r, 0)),
        out_shape=jax.ShapeDtypeStruct((nb2, n2, 16), jnp.float32),
    )


def _tc_pool(n2, f, g, nout):
    nb = f // 16
    ng = n2 // RB

    def body(scat_ref, hp_ref, dis_ref, b_ref, batch_ref, wp_ref, bp_ref,
             pol_ref, sums_ref, cnts_ref):
        r = pl.program_id(0)

        @pl.when(r == 0)
        def _():
            sums_ref[...] = jnp.zeros((g, f), jnp.float32)
            cnts_ref[...] = jnp.zeros((g, f), jnp.float32)
            pol_ref[...] = jnp.zeros((g, nout), jnp.float32)

        dis = dis_ref[...]
        parts = [dis * (scat_ref[i] + hp_ref[i]) for i in range(nb)]
        m = jnp.concatenate(parts, axis=-1)
        h3 = jnp.maximum(m + b_ref[...], 0.0)
        bv = batch_ref[0, :]
        oh = (bv[:, None] == lax.broadcasted_iota(jnp.int32, (RB, g), 1))
        ohf = oh.astype(jnp.float32)
        sums_ref[...] += lax.dot_general(
            ohf, h3, (((0,), (0,)), ((), ())),
            preferred_element_type=jnp.float32,
        )
        cnts_ref[...] += lax.dot_general(
            ohf, jnp.ones((RB, f), jnp.float32), (((0,), (0,)), ((), ())),
            preferred_element_type=jnp.float32,
        )

        @pl.when(r == ng - 1)
        def _():
            pooled = sums_ref[...] / jnp.maximum(cnts_ref[...], 1.0)
            pol_ref[...] = (
                jnp.dot(pooled, wp_ref[...], preferred_element_type=jnp.float32)
                + bp_ref[...]
            )

    return pl.pallas_call(
        body,
        grid=(ng,),
        in_specs=[
            pl.BlockSpec((nb, RB, 16), lambda r: (0, r, 0)),
            pl.BlockSpec((nb, RB, 16), lambda r: (0, r, 0)),
            pl.BlockSpec((RB, 1), lambda r: (r, 0)),
            pl.BlockSpec((1, f), lambda r: (0, 0)),
            pl.BlockSpec((1, RB), lambda r: (0, r)),
            pl.BlockSpec((f, nout), lambda r: (0, 0)),
            pl.BlockSpec((1, nout), lambda r: (0, 0)),
        ],
        out_specs=pl.BlockSpec((g, nout), lambda r: (0, 0)),
        out_shape=jax.ShapeDtypeStruct((g, nout), jnp.float32),
        scratch_shapes=[
            pltpu.VMEM((g, f), jnp.float32),
            pltpu.VMEM((g, f), jnp.float32),
        ],
    )


def kernel(x, edge_index, edge_weight, batch, W1, b1, W2, b2, W3, b3, Wp, bp):
    n, fin = x.shape
    e = edge_weight.shape[0]
    f1 = W1.shape[1]
    f2 = W2.shape[1]
    f3 = W3.shape[1]
    g = 64
    nout = Wp.shape[1]

    n2 = ((n + NS * RB - 1) // (NS * RB)) * (NS * RB)
    estep = NS * CH
    epad = ((e + estep - 1) // estep) * estep

    xp = jnp.pad(x, ((0, n2 - n), (0, 0)))
    s = jnp.pad(edge_index[0], (0, epad - e)).reshape(epad // CW, CW)
    d = jnp.pad(edge_index[1], (0, epad - e)).reshape(epad // CW, CW)
    w = jnp.pad(edge_weight, (0, epad - e)).reshape(epad // CW, CW)
    batchp = jnp.pad(batch, (0, n2 - n), constant_values=g).reshape(1, n2)

    degp = _make_deg(n2, epad)(d, w)
    dis, hp1 = _tc_prep(n2, f1)(degp.reshape(NC, n2), xp, W1)

    agg1 = _make_agg(f1 // 16, n2, epad)
    scat1 = agg1(hp1.reshape(f1 // 16 * n2, 16), s, d, w)
    hp2 = _tc_layer(n2, f1, f2)(
        scat1.reshape(f1 // 16, n2, 16), hp1, dis, b1.reshape(1, f1), W2
    )

    agg2 = _make_agg(f2 // 16, n2, epad)
    scat2 = agg2(hp2.reshape(f2 // 16 * n2, 16), s, d, w)
    hp3 = _tc_layer(n2, f2, f3)(
        scat2.reshape(f2 // 16, n2, 16), hp2, dis, b2.reshape(1, f2), W3
    )

    agg3 = _make_agg(f3 // 16, n2, epad)
    scat3 = agg3(hp3.reshape(f3 // 16 * n2, 16), s, d, w)

    pol = _tc_pool(n2, f3, g, nout)(
        scat3.reshape(f3 // 16, n2, 16), hp3, dis, b3.reshape(1, f3),
        batchp, Wp, bp.reshape(1, nout),
    )
    return pol


# speedup vs baseline: 3.1353x; 1.6389x over previous
import functools

import jax
import jax.numpy as jnp
from jax import lax
from jax.experimental import pallas as pl
from jax.experimental.pallas import tpu as pltpu
from jax.experimental.pallas import tpu_sc as plsc

NC = 2
NS = 16
RB = 512
CH = 512
CW = 128

_SC_PARAMS = pltpu.CompilerParams(
    needs_layout_passes=False, use_tc_tiling_on_sc=False
)


def _sc_mesh():
    return plsc.VectorSubcoreMesh(
        core_axis_name="c", subcore_axis_name="s", num_cores=NC, num_subcores=NS
    )


def _make_deg(n2, epad):
    epc = epad // (NC * NS)
    nchunks = epc // CH
    nsl = n2 // NS

    @functools.partial(
        pl.kernel,
        out_type=jax.ShapeDtypeStruct((NC * n2,), jnp.float32),
        mesh=_sc_mesh(),
        scratch_types=[
            pltpu.VMEM_SHARED((n2,), jnp.float32),
            pltpu.VMEM((CH // CW, CW), jnp.int32),
            pltpu.VMEM((CH // CW, CW), jnp.float32),
            pltpu.VMEM((nsl,), jnp.float32),
        ],
        compiler_params=_SC_PARAMS,
    )
    def k(d_idx, w, out, acc, dbuf, wbuf, zbuf):
        c = lax.axis_index("c")
        sid = lax.axis_index("s")
        zv = jnp.zeros((16,), jnp.float32)

        def zfill(i, _):
            zbuf[pl.ds(i * 16, 16)] = zv
            return 0

        lax.fori_loop(0, nsl // 16, zfill, 0)
        pltpu.sync_copy(zbuf, acc.at[pl.ds(sid * nsl, nsl)])
        plsc.subcore_barrier()

        ebase = (c * NS + sid) * epc

        def chunk(ci, _):
            base = ebase + ci * CH
            pltpu.sync_copy(d_idx.at[pl.ds(base // CW, CH // CW)], dbuf)
            pltpu.sync_copy(w.at[pl.ds(base // CW, CH // CW)], wbuf)
            for j in range(CH // CW):
                pltpu.sync_copy(wbuf.at[j], acc.at[dbuf.at[j]], add=True)
            return 0

        lax.fori_loop(0, nchunks, chunk, 0)
        plsc.subcore_barrier()
        lo = sid * nsl
        pltpu.sync_copy(acc.at[pl.ds(lo, nsl)], out.at[pl.ds(c * n2 + lo, nsl)])

    return k


def _make_bin(n2, epad):
    h2 = n2 // NC
    eps32 = epad // (NC * NS)
    rr = eps32 // CW
    orows = epad // CW

    @functools.partial(
        pl.kernel,
        out_type=[
            jax.ShapeDtypeStruct((2 * orows, CW), jnp.int32),
            jax.ShapeDtypeStruct((2 * orows, CW), jnp.int32),
            jax.ShapeDtypeStruct((2 * orows, CW), jnp.float32),
            jax.ShapeDtypeStruct((2 * NC * NS * 16,), jnp.int32),
        ],
        mesh=_sc_mesh(),
        scratch_types=[
            pltpu.VMEM((4, CW), jnp.int32),
            pltpu.VMEM((4, CW), jnp.int32),
            pltpu.VMEM((4, CW), jnp.float32),
            pltpu.VMEM((2, 288), jnp.int32),
            pltpu.VMEM((2, 288), jnp.int32),
            pltpu.VMEM((2, 288), jnp.float32),
            pltpu.VMEM((16,), jnp.int32),
        ],
        compiler_params=_SC_PARAMS,
    )
    def k(s_in, d_in, w_in, sB, dB, wB, cnts, sI, dI, wI, sSt, dSt, wSt, cbuf):
        c = lax.axis_index("c")
        sid = lax.axis_index("s")
        wid = c * NS + sid
        inrow = wid * rr
        obase0 = wid * rr
        obase1 = orows + wid * rr
        iota = lax.iota(jnp.int32, 16)

        def flush(b, obase, fp, done):
            def do_flush(_):
                row = obase + done
                pltpu.sync_copy(sSt.at[b, pl.ds(0, CW)], sB.at[row])
                pltpu.sync_copy(dSt.at[b, pl.ds(0, CW)], dB.at[row])
                pltpu.sync_copy(wSt.at[b, pl.ds(0, CW)], wB.at[row])
                for t in range(CW // 16):
                    src = pl.ds(CW + t * 16, 16)
                    dst = pl.ds(t * 16, 16)
                    sSt[b, dst] = sSt[b, src]
                    dSt[b, dst] = dSt[b, src]
                    wSt[b, dst] = wSt[b, src]
                return fp - CW, done + 1

            return lax.cond(fp >= CW, do_flush, lambda _: (fp, done), 0)

        def row_body(ri, carry):
            fpl, dl, fph, dh = carry
            j = ri % 4

            @pl.when(j == 0)
            def _():
                pltpu.sync_copy(s_in.at[pl.ds(inrow + ri, 4)], sI)
                pltpu.sync_copy(d_in.at[pl.ds(inrow + ri, 4)], dI)
                pltpu.sync_copy(w_in.at[pl.ds(inrow + ri, 4)], wI)

            def vreg_body(kk, car):
                fpl, fph = car
                sl = pl.ds(kk * 16, 16)
                sv = sI[j, sl]
                dv = dI[j, sl]
                wv = wI[j, sl]
                m = dv < h2
                cl = jnp.sum(m.astype(jnp.int32))
                plsc.store_compressed(sSt.at[0, pl.ds(fpl, 16)], sv, mask=m)
                plsc.store_compressed(dSt.at[0, pl.ds(fpl, 16)], dv, mask=m)
                plsc.store_compressed(wSt.at[0, pl.ds(fpl, 16)], wv, mask=m)
                mh = jnp.logical_not(m)
                plsc.store_compressed(sSt.at[1, pl.ds(fph, 16)], sv, mask=mh)
                plsc.store_compressed(dSt.at[1, pl.ds(fph, 16)], dv - h2,
                                      mask=mh)
                plsc.store_compressed(wSt.at[1, pl.ds(fph, 16)], wv, mask=mh)
                return fpl + cl, fph + (16 - cl)

            fpl, fph = lax.fori_loop(0, CW // 16, vreg_body, (fpl, fph))
            fpl, dl = flush(0, obase0, fpl, dl)
            fph, dh = flush(1, obase1, fph, dh)
            return fpl, dl, fph, dh

        z4 = (jnp.int32(0),) * 4
        fpl, dl, fph, dh = lax.fori_loop(0, rr, row_body, z4)

        def finish(b, obase, base_node, fp, done):
            def pad_row(t, _):
                nulls = base_node + t * 16 + iota
                pos = pl.ds(fp + t * 16, 16)
                sSt[b, pos] = nulls
                dSt[b, pos] = nulls
                wSt[b, pos] = jnp.zeros((16,), jnp.float32)
                return 0

            lax.fori_loop(0, CW // 16, pad_row, 0)
            fp2, done2 = lax.cond(
                fp > 0,
                lambda _: flush(b, obase, fp + CW, done),
                lambda _: (fp, done),
                0,
            )

            def null_row(t, _):
                nulls = base_node + t * 16 + iota
                pos = pl.ds(t * 16, 16)
                sSt[b, pos] = nulls
                dSt[b, pos] = nulls
                wSt[b, pos] = jnp.zeros((16,), jnp.float32)
                return 0

            lax.fori_loop(0, CW // 16, null_row, 0)

            def pad4(t, d2):
                def do(_):
                    row = obase + d2
                    pltpu.sync_copy(sSt.at[b, pl.ds(0, CW)], sB.at[row])
                    pltpu.sync_copy(dSt.at[b, pl.ds(0, CW)], dB.at[row])
                    pltpu.sync_copy(wSt.at[b, pl.ds(0, CW)], wB.at[row])
                    return d2 + 1

                return lax.cond(d2 % 4 != 0, do, lambda _: d2, 0)

            done3 = lax.fori_loop(0, 3, pad4, done2)
            cbuf[...] = jnp.broadcast_to(done3 // 4, (16,)).astype(jnp.int32)
            pltpu.sync_copy(
                cbuf, cnts.at[pl.ds((b * NC * NS + wid) * 16, 16)]
            )

        finish(0, obase0, wid * 128, fpl, dl)
        finish(1, obase1, wid * 128, fph, dh)

    return k


def _make_agg(nb, n2, epad):
    h2 = n2 // NC
    eps32 = epad // (NC * NS)
    rr = eps32 // CW
    orows = epad // CW
    nsl = h2 // NS

    @functools.partial(
        pl.kernel,
        out_type=jax.ShapeDtypeStruct((nb * n2, 16), jnp.float32),
        mesh=_sc_mesh(),
        scratch_types=[
            pltpu.VMEM_SHARED((h2, 16), jnp.float32),
            pltpu.VMEM((CH // CW, CW), jnp.int32),
            pltpu.VMEM((CH // CW, CW), jnp.int32),
            pltpu.VMEM((CH // CW, CW), jnp.float32),
            pltpu.VMEM((CH, 16), jnp.float32),
            pltpu.VMEM((nsl, 16), jnp.float32),
            pltpu.VMEM((16,), jnp.int32),
            pltpu.SemaphoreType.DMA,
        ],
        compiler_params=_SC_PARAMS,
    )
    def k(hp, sB, dB, wB, cnts, out, acc, sbuf, dbuf, wbuf, rows, zrows,
          cbuf, sem):
        c = lax.axis_index("c")
        sid = lax.axis_index("s")
        zv = jnp.zeros((16,), jnp.float32)

        def zfill(i, _):
            zrows[i] = zv
            return 0

        lax.fori_loop(0, nsl, zfill, 0)

        r0 = sid * 2
        r1 = sid * 2 + 1
        pltpu.sync_copy(cnts.at[pl.ds((c * NC * NS + r0) * 16, 16)], cbuf)
        nch0 = jnp.max(cbuf[...])
        pltpu.sync_copy(cnts.at[pl.ds((c * NC * NS + r1) * 16, 16)], cbuf)
        nch1 = jnp.max(cbuf[...])

        def do_fb(fb):
            pltpu.sync_copy(zrows, acc.at[pl.ds(sid * nsl, nsl)])
            plsc.subcore_barrier()
            off = fb * n2

            def chunk(ci, regbase):
                rows4 = regbase + ci * (CH // CW)
                pltpu.sync_copy(sB.at[pl.ds(rows4, CH // CW)], sbuf)
                pltpu.sync_copy(dB.at[pl.ds(rows4, CH // CW)], dbuf)
                pltpu.sync_copy(wB.at[pl.ds(rows4, CH // CW)], wbuf)

                def adj(j, _):
                    for kk in range(CW // 16):
                        sl = pl.ds(kk * 16, 16)
                        sbuf[j, sl] = sbuf[j, sl] + off
                    return 0

                lax.fori_loop(0, CH // CW, adj, 0)

                cps = []
                for j in range(CH // CW):
                    cps.append(
                        pltpu.async_copy(
                            hp.at[sbuf.at[j]],
                            rows.at[pl.ds(j * CW, CW)],
                            sem,
                        )
                    )
                for cp in cps:
                    cp.wait()

                def scale(j, _):
                    for kk in range(16):
                        i = j * 16 + kk
                        bc = plsc.load_gather(
                            wbuf,
                            [
                                jnp.full((16,), i // CW, jnp.int32),
                                jnp.full((16,), i % CW, jnp.int32),
                            ],
                        )
                        rows[i] = rows[i] * bc
                    return 0

                lax.fori_loop(0, CH // 16, scale, 0)

                for j in range(CH // CW):
                    pltpu.sync_copy(
                        rows.at[pl.ds(j * CW, CW)],
                        acc.at[dbuf.at[j]],
                        add=True,
                    )
                return regbase

            base0 = c * orows + r0 * rr
            base1 = c * orows + r1 * rr
            lax.fori_loop(0, nch0, chunk, base0)
            lax.fori_loop(0, nch1, chunk, base1)
            plsc.subcore_barrier()
            lo = sid * nsl
            pltpu.sync_copy(
                acc.at[pl.ds(lo, nsl)],
                out.at[pl.ds(fb * n2 + c * h2 + lo, nsl)],
            )
            plsc.subcore_barrier()

        for fb in range(nb):
            do_fb(fb)

    return k


def _tc_prep(n2, f1):
    nb = f1 // 16
    ng = n2 // RB

    def body(deg_ref, x_ref, w1_ref, dis_ref, hp_ref):
        deg = deg_ref[0, :] + deg_ref[1, :] + 1.0
        dis = lax.rsqrt(jnp.maximum(deg, 1e-12))
        h = jnp.dot(x_ref[...], w1_ref[...], preferred_element_type=jnp.float32)
        hp = h * dis[:, None]
        dis_ref[...] = dis[:, None]
        for i in range(nb):
            hp_ref[i] = hp[:, i * 16:(i + 1) * 16]

    return pl.pallas_call(
        body,
        grid=(ng,),
        in_specs=[
            pl.BlockSpec((2, RB), lambda r: (0, r)),
            pl.BlockSpec((RB, 32), lambda r: (r, 0)),
            pl.BlockSpec((32, f1), lambda r: (0, 0)),
        ],
        out_specs=[
            pl.BlockSpec((RB, 1), lambda r: (r, 0)),
            pl.BlockSpec((nb, RB, 16), lambda r: (0, r, 0)),
        ],
        out_shape=[
            jax.ShapeDtypeStruct((n2, 1), jnp.float32),
            jax.ShapeDtypeStruct((nb, n2, 16), jnp.float32),
        ],
    )


def _tc_layer(n2, f, f2):
    nb, nb2 = f // 16, f2 // 16
    ng = n2 // RB

    def body(scat_ref, hp_ref, dis_ref, b_ref, w_ref, hp2_ref):
        dis = dis_ref[...]
        parts = [
            dis * (scat_ref[i] + hp_ref[i]) for i in range(nb)
        ]
        m = jnp.concatenate(parts, axis=-1)
        act = jnp.maximum(m + b_ref[...], 0.0)
        h2 = jnp.dot(act, w_ref[...], preferred_element_type=jnp.float32)
        hp2 = h2 * dis
        for i in range(nb2):
            hp2_ref[i] = hp2[:, i * 16:(i + 1) * 16]

    return pl.pallas_call(
        body,
        grid=(ng,),
        in_specs=[
            pl.BlockSpec((nb, RB, 16), lambda r: (0, r, 0)),
            pl.BlockSpec((nb, RB, 16), lambda r: (0, r, 0)),
            pl.BlockSpec((RB, 1), lambda r: (r, 0)),
            pl.BlockSpec((1, f), lambda r: (0, 0)),
            pl.BlockSpec((f, f2), lambda r: (0, 0)),
        ],
        out_specs=pl.BlockSpec((nb2, RB, 16), lambda r: (0, r, 0)),
        out_shape=jax.ShapeDtypeStruct((nb2, n2, 16), jnp.float32),
    )


def _tc_pool(n2, f, g, nout):
    nb = f // 16
    ng = n2 // RB

    def body(scat_ref, hp_ref, dis_ref, b_ref, batch_ref, wp_ref, bp_ref,
             pol_ref, sums_ref, cnts_ref):
        r = pl.program_id(0)

        @pl.when(r == 0)
        def _():
            sums_ref[...] = jnp.zeros((g, f), jnp.float32)
            cnts_ref[...] = jnp.zeros((g, f), jnp.float32)
            pol_ref[...] = jnp.zeros((g, nout), jnp.float32)

        dis = dis_ref[...]
        parts = [dis * (scat_ref[i] + hp_ref[i]) for i in range(nb)]
        m = jnp.concatenate(parts, axis=-1)
        h3 = jnp.maximum(m + b_ref[...], 0.0)
        bv = batch_ref[0, :]
        oh = (bv[:, None] == lax.broadcasted_iota(jnp.int32, (RB, g), 1))
        ohf = oh.astype(jnp.float32)
        sums_ref[...] += lax.dot_general(
            ohf, h3, (((0,), (0,)), ((), ())),
            preferred_element_type=jnp.float32,
        )
        cnts_ref[...] += lax.dot_general(
            ohf, jnp.ones((RB, f), jnp.float32), (((0,), (0,)), ((), ())),
            preferred_element_type=jnp.float32,
        )

        @pl.when(r == ng - 1)
        def _():
            pooled = sums_ref[...] / jnp.maximum(cnts_ref[...], 1.0)
            pol_ref[...] = (
                jnp.dot(pooled, wp_ref[...], preferred_element_type=jnp.float32)
                + bp_ref[...]
            )

    return pl.pallas_call(
        body,
        grid=(ng,),
        in_specs=[
            pl.BlockSpec((nb, RB, 16), lambda r: (0, r, 0)),
            pl.BlockSpec((nb, RB, 16), lambda r: (0, r, 0)),
            pl.BlockSpec((RB, 1), lambda r: (r, 0)),
            pl.BlockSpec((1, f), lambda r: (0, 0)),
            pl.BlockSpec((1, RB), lambda r: (0, r)),
            pl.BlockSpec((f, nout), lambda r: (0, 0)),
            pl.BlockSpec((1, nout), lambda r: (0, 0)),
        ],
        out_specs=pl.BlockSpec((g, nout), lambda r: (0, 0)),
        out_shape=jax.ShapeDtypeStruct((g, nout), jnp.float32),
        scratch_shapes=[
            pltpu.VMEM((g, f), jnp.float32),
            pltpu.VMEM((g, f), jnp.float32),
        ],
    )


def kernel(x, edge_index, edge_weight, batch, W1, b1, W2, b2, W3, b3, Wp, bp):
    n, fin = x.shape
    e = edge_weight.shape[0]
    f1 = W1.shape[1]
    f2 = W2.shape[1]
    f3 = W3.shape[1]
    g = 64
    nout = Wp.shape[1]

    n2 = ((n + NS * RB - 1) // (NS * RB)) * (NS * RB)
    estep = NS * CH
    epad = ((e + estep - 1) // estep) * estep

    xp = jnp.pad(x, ((0, n2 - n), (0, 0)))
    s = jnp.pad(edge_index[0], (0, epad - e)).reshape(epad // CW, CW)
    d = jnp.pad(edge_index[1], (0, epad - e)).reshape(epad // CW, CW)
    w = jnp.pad(edge_weight, (0, epad - e)).reshape(epad // CW, CW)
    batchp = jnp.pad(batch, (0, n2 - n), constant_values=g).reshape(1, n2)

    degp = _make_deg(n2, epad)(d, w)
    dis, hp1 = _tc_prep(n2, f1)(degp.reshape(NC, n2), xp, W1)

    sB, dB, wB, cnts = _make_bin(n2, epad)(s, d, w)

    agg1 = _make_agg(f1 // 16, n2, epad)
    scat1 = agg1(hp1.reshape(f1 // 16 * n2, 16), sB, dB, wB, cnts)
    hp2 = _tc_layer(n2, f1, f2)(
        scat1.reshape(f1 // 16, n2, 16), hp1, dis, b1.reshape(1, f1), W2
    )

    agg2 = _make_agg(f2 // 16, n2, epad)
    scat2 = agg2(hp2.reshape(f2 // 16 * n2, 16), sB, dB, wB, cnts)
    hp3 = _tc_layer(n2, f2, f3)(
        scat2.reshape(f2 // 16, n2, 16), hp2, dis, b2.reshape(1, f2), W3
    )

    agg3 = _make_agg(f3 // 16, n2, epad)
    scat3 = agg3(hp3.reshape(f3 // 16 * n2, 16), sB, dB, wB, cnts)

    pol = _tc_pool(n2, f3, g, nout)(
        scat3.reshape(f3 // 16, n2, 16), hp3, dis, b3.reshape(1, f3),
        batchp, Wp, bp.reshape(1, nout),
    )
    return pol


# speedup vs baseline: 5.5345x; 1.7652x over previous
import functools

import jax
import jax.numpy as jnp
from jax import lax
from jax.experimental import pallas as pl
from jax.experimental.pallas import tpu as pltpu
from jax.experimental.pallas import tpu_sc as plsc

NC = 2
NS = 16
RB = 512
CW = 128
CH = 512
CHW = 256

_SC_PARAMS = pltpu.CompilerParams(
    needs_layout_passes=False, use_tc_tiling_on_sc=False
)


def _sc_mesh():
    return plsc.VectorSubcoreMesh(
        core_axis_name="c", subcore_axis_name="s", num_cores=NC, num_subcores=NS
    )


def _make_deg(n2, epad):
    epc = epad // (NC * NS)
    nchunks = epc // CH
    nsl = n2 // NS

    @functools.partial(
        pl.kernel,
        out_type=jax.ShapeDtypeStruct((NC * n2,), jnp.float32),
        mesh=_sc_mesh(),
        scratch_types=[
            pltpu.VMEM_SHARED((n2,), jnp.float32),
            pltpu.VMEM((CH // CW, CW), jnp.int32),
            pltpu.VMEM((CH // CW, CW), jnp.float32),
            pltpu.VMEM((nsl,), jnp.float32),
        ],
        compiler_params=_SC_PARAMS,
    )
    def k(d_idx, w, out, acc, dbuf, wbuf, zbuf):
        c = lax.axis_index("c")
        sid = lax.axis_index("s")
        zv = jnp.zeros((16,), jnp.float32)

        def zfill(i, _):
            zbuf[pl.ds(i * 16, 16)] = zv
            return 0

        lax.fori_loop(0, nsl // 16, zfill, 0)
        pltpu.sync_copy(zbuf, acc.at[pl.ds(sid * nsl, nsl)])
        plsc.subcore_barrier()

        ebase = (c * NS + sid) * epc

        def chunk(ci, _):
            base = ebase + ci * CH
            pltpu.sync_copy(d_idx.at[pl.ds(base // CW, CH // CW)], dbuf)
            pltpu.sync_copy(w.at[pl.ds(base // CW, CH // CW)], wbuf)
            for j in range(CH // CW):
                pltpu.sync_copy(wbuf.at[j], acc.at[dbuf.at[j]], add=True)
            return 0

        lax.fori_loop(0, nchunks, chunk, 0)
        plsc.subcore_barrier()
        lo = sid * nsl
        pltpu.sync_copy(acc.at[pl.ds(lo, nsl)], out.at[pl.ds(c * n2 + lo, nsl)])

    return k


def _make_bin(n2, epad, nbuk):
    br = n2 // nbuk
    eps32 = epad // (NC * NS)
    rr = eps32 // CW
    orows = epad // CW

    @functools.partial(
        pl.kernel,
        out_type=[
            jax.ShapeDtypeStruct((nbuk * orows, CW), jnp.int32),
            jax.ShapeDtypeStruct((nbuk * orows, CW), jnp.int32),
            jax.ShapeDtypeStruct((nbuk * orows, CW), jnp.float32),
            jax.ShapeDtypeStruct((nbuk * NC * NS * 16,), jnp.int32),
        ],
        mesh=_sc_mesh(),
        scratch_types=[
            pltpu.VMEM((4, CW), jnp.int32),
            pltpu.VMEM((4, CW), jnp.int32),
            pltpu.VMEM((4, CW), jnp.float32),
            pltpu.VMEM((nbuk, 288), jnp.int32),
            pltpu.VMEM((nbuk, 288), jnp.int32),
            pltpu.VMEM((nbuk, 288), jnp.float32),
            pltpu.VMEM((16,), jnp.int32),
            pltpu.VMEM((nbuk, 16), jnp.int32),
            pltpu.VMEM((nbuk, 16), jnp.int32),
        ],
        compiler_params=_SC_PARAMS,
    )
    def k(s_in, d_in, w_in, sB, dB, wB, cnts, sI, dI, wI, sSt, dSt, wSt, cbuf,
          fpbuf, dnbuf):
        c = lax.axis_index("c")
        sid = lax.axis_index("s")
        wid = c * NS + sid
        inrow = wid * rr
        iota = lax.iota(jnp.int32, 16)
        nullbase = (wid % 16) * 128

        def obase(b):
            return b * orows + wid * rr

        def flush(b, fp, done):
            def do_flush(_):
                row = obase(b) + done
                pltpu.sync_copy(sSt.at[b, pl.ds(0, CW)], sB.at[row])
                pltpu.sync_copy(dSt.at[b, pl.ds(0, CW)], dB.at[row])
                pltpu.sync_copy(wSt.at[b, pl.ds(0, CW)], wB.at[row])

                def mv(t, _):
                    src = pl.ds(CW + t * 16, 16)
                    dst = pl.ds(t * 16, 16)
                    sSt[b, dst] = sSt[b, src]
                    dSt[b, dst] = dSt[b, src]
                    wSt[b, dst] = wSt[b, src]
                    return 0

                lax.fori_loop(0, CW // 16, mv, 0)
                return fp - CW, done + 1

            return lax.cond(fp >= CW, do_flush, lambda _: (fp, done), 0)

        def row_body(ri, carry):
            fps, dones = carry
            j = ri % 4

            @pl.when(j == 0)
            def _():
                pltpu.sync_copy(s_in.at[pl.ds(inrow + ri, 4)], sI)
                pltpu.sync_copy(d_in.at[pl.ds(inrow + ri, 4)], dI)
                pltpu.sync_copy(w_in.at[pl.ds(inrow + ri, 4)], wI)

            def vreg_body(kk, car):
                fps = car
                sl = pl.ds(kk * 16, 16)
                sv = sI[j, sl]
                dv = dI[j, sl]
                wv = wI[j, sl]
                bid = dv // br
                dloc = dv - bid * br
                out_fps = []
                for b in range(nbuk):
                    m = bid == b
                    cl = jnp.sum(m.astype(jnp.int32))
                    fp = fps[b]
                    plsc.store_compressed(sSt.at[b, pl.ds(fp, 16)], sv, mask=m)
                    plsc.store_compressed(dSt.at[b, pl.ds(fp, 16)], dloc,
                                          mask=m)
                    plsc.store_compressed(wSt.at[b, pl.ds(fp, 16)], wv, mask=m)
                    out_fps.append(fp + cl)
                return tuple(out_fps)

            fps = lax.fori_loop(0, CW // 16, vreg_body, fps)
            new_fps, new_dones = [], []
            for b in range(nbuk):
                fp, dn = flush(b, fps[b], dones[b])
                new_fps.append(fp)
                new_dones.append(dn)
            return tuple(new_fps), tuple(new_dones)

        z = tuple(jnp.int32(0) for _ in range(nbuk))
        fps, dones = lax.fori_loop(0, rr, row_body, (z, z))
        for b in range(nbuk):
            fpbuf[b] = jnp.broadcast_to(fps[b], (16,)).astype(jnp.int32)
            dnbuf[b] = jnp.broadcast_to(dones[b], (16,)).astype(jnp.int32)

        def finish(b, fp, done):
            def pad_row(t, _):
                nulls = nullbase + t * 16 + iota
                pos = pl.ds(fp + t * 16, 16)
                sSt[b, pos] = nulls
                dSt[b, pos] = nulls
                wSt[b, pos] = jnp.zeros((16,), jnp.float32)
                return 0

            lax.fori_loop(0, CW // 16, pad_row, 0)
            fp2, done2 = lax.cond(
                fp > 0,
                lambda _: flush(b, fp + CW, done),
                lambda _: (fp, done),
                0,
            )

            def null_row(t, _):
                nulls = nullbase + t * 16 + iota
                pos = pl.ds(t * 16, 16)
                sSt[b, pos] = nulls
                dSt[b, pos] = nulls
                wSt[b, pos] = jnp.zeros((16,), jnp.float32)
                return 0

            lax.fori_loop(0, CW // 16, null_row, 0)

            def pad2(t, d2):
                def do(_):
                    row = obase(b) + d2
                    pltpu.sync_copy(sSt.at[b, pl.ds(0, CW)], sB.at[row])
                    pltpu.sync_copy(dSt.at[b, pl.ds(0, CW)], dB.at[row])
                    pltpu.sync_copy(wSt.at[b, pl.ds(0, CW)], wB.at[row])
                    return d2 + 1

                return lax.cond(d2 % (CHW // CW) != 0, do, lambda _: d2, 0)

            done3 = lax.fori_loop(0, (CHW // CW) - 1, pad2, done2)
            cbuf[...] = jnp.broadcast_to(
                done3 // (CHW // CW), (16,)
            ).astype(jnp.int32)
            pltpu.sync_copy(
                cbuf, cnts.at[pl.ds((b * NC * NS + wid) * 16, 16)]
            )

        def finish_b(b, _):
            fp = jnp.max(fpbuf[b])
            done = jnp.max(dnbuf[b])
            finish(b, fp, done)
            return 0

        lax.fori_loop(0, nbuk, finish_b, 0)

    return k


def _make_agg(f, n2, epad, nbuk):
    br = n2 // nbuk
    eps32 = epad // (NC * NS)
    rr = eps32 // CW
    orows = epad // CW
    nsl = br // NS
    zr = 32

    @functools.partial(
        pl.kernel,
        out_type=jax.ShapeDtypeStruct((n2, f), jnp.float32),
        mesh=_sc_mesh(),
        scratch_types=[
            pltpu.VMEM_SHARED((br, f), jnp.float32),
            pltpu.VMEM((CHW // CW, CW), jnp.int32),
            pltpu.VMEM((CHW // CW, CW), jnp.int32),
            pltpu.VMEM((CHW // CW, CW), jnp.float32),
            pltpu.VMEM((CHW, f), jnp.float32),
            pltpu.VMEM((zr, f), jnp.float32),
            pltpu.VMEM((16,), jnp.int32),
            pltpu.SemaphoreType.DMA,
        ],
        compiler_params=_SC_PARAMS,
    )
    def k(hp, sB, dB, wB, cnts, out, acc, sbuf, dbuf, wbuf, rows, zrows,
          cbuf, sem):
        c = lax.axis_index("c")
        sid = lax.axis_index("s")
        zv = jnp.zeros((16,), jnp.float32)

        def zfill(i, _):
            for q in range(f // 16):
                zrows[i, pl.ds(q * 16, 16)] = zv
            return 0

        lax.fori_loop(0, zr, zfill, 0)

        r0 = sid * 2
        r1 = sid * 2 + 1
        dnums = lax.GatherDimensionNumbers(
            offset_dims=(), collapsed_slice_dims=(0,), start_index_map=(0,)
        )

        def chunk(ci, regbase):
            rows2 = regbase + ci * (CHW // CW)
            pltpu.sync_copy(sB.at[pl.ds(rows2, CHW // CW)], sbuf)
            pltpu.sync_copy(dB.at[pl.ds(rows2, CHW // CW)], dbuf)
            pltpu.sync_copy(wB.at[pl.ds(rows2, CHW // CW)], wbuf)
            cps = []
            for j in range(CHW // CW):
                cps.append(
                    pltpu.async_copy(
                        hp.at[sbuf.at[j]], rows.at[pl.ds(j * CW, CW)], sem
                    )
                )
            for cp in cps:
                cp.wait()

            def scale(j, _):
                wv = wbuf[j // 8, pl.ds((j % 8) * 16, 16)]
                for kk in range(16):
                    i = j * 16 + kk
                    idx = jnp.full((16, 1), kk, jnp.int32)
                    bc = lax.gather(
                        wv, idx, dnums, (1,),
                        mode=lax.GatherScatterMode.PROMISE_IN_BOUNDS,
                    )
                    for q in range(f // 16):
                        sl = pl.ds(q * 16, 16)
                        rows[i, sl] = rows[i, sl] * bc
                return 0

            lax.fori_loop(0, CHW // 16, scale, 0)
            for j in range(CHW // CW):
                pltpu.sync_copy(
                    rows.at[pl.ds(j * CW, CW)], acc.at[dbuf.at[j]], add=True
                )
            return regbase

        def do_bucket(bi, _):
            b = c + NC * bi
            for t in range(nsl // zr):
                pltpu.sync_copy(
                    zrows, acc.at[pl.ds(sid * nsl + t * zr, zr)]
                )
            plsc.subcore_barrier()

            for r in (r0, r1):
                pltpu.sync_copy(
                    cnts.at[pl.ds((b * NC * NS + r) * 16, 16)], cbuf
                )
                nch = jnp.max(cbuf[...])
                regbase = b * orows + r * rr
                lax.fori_loop(0, nch, chunk, regbase)

            plsc.subcore_barrier()
            lo = sid * nsl
            pltpu.sync_copy(
                acc.at[pl.ds(lo, nsl)], out.at[pl.ds(b * br + lo, nsl)]
            )
            plsc.subcore_barrier()
            return 0

        lax.fori_loop(0, nbuk // NC, do_bucket, 0)

    return k


def _tc_prep(n2, f1):
    ng = n2 // RB

    def body(deg_ref, x_ref, w1_ref, dis_ref, hp_ref):
        deg = deg_ref[0, :] + deg_ref[1, :] + 1.0
        dis = lax.rsqrt(jnp.maximum(deg, 1e-12))
        h = jnp.dot(x_ref[...], w1_ref[...], preferred_element_type=jnp.float32)
        dis_ref[...] = dis[:, None]
        hp_ref[...] = h * dis[:, None]

    return pl.pallas_call(
        body,
        grid=(ng,),
        in_specs=[
            pl.BlockSpec((2, RB), lambda r: (0, r)),
            pl.BlockSpec((RB, 32), lambda r: (r, 0)),
            pl.BlockSpec((32, f1), lambda r: (0, 0)),
        ],
        out_specs=[
            pl.BlockSpec((RB, 1), lambda r: (r, 0)),
            pl.BlockSpec((RB, f1), lambda r: (r, 0)),
        ],
        out_shape=[
            jax.ShapeDtypeStruct((n2, 1), jnp.float32),
            jax.ShapeDtypeStruct((n2, f1), jnp.float32),
        ],
    )


def _tc_layer(n2, f, f2):
    ng = n2 // RB

    def body(scat_ref, hp_ref, dis_ref, b_ref, w_ref, hp2_ref):
        dis = dis_ref[...]
        m = dis * (scat_ref[...] + hp_ref[...])
        act = jnp.maximum(m + b_ref[...], 0.0)
        h2 = jnp.dot(act, w_ref[...], preferred_element_type=jnp.float32)
        hp2_ref[...] = h2 * dis

    return pl.pallas_call(
        body,
        grid=(ng,),
        in_specs=[
            pl.BlockSpec((RB, f), lambda r: (r, 0)),
            pl.BlockSpec((RB, f), lambda r: (r, 0)),
            pl.BlockSpec((RB, 1), lambda r: (r, 0)),
            pl.BlockSpec((1, f), lambda r: (0, 0)),
            pl.BlockSpec((f, f2), lambda r: (0, 0)),
        ],
        out_specs=pl.BlockSpec((RB, f2), lambda r: (r, 0)),
        out_shape=jax.ShapeDtypeStruct((n2, f2), jnp.float32),
    )


def _tc_pool(n2, f, g, nout):
    ng = n2 // RB

    def body(scat_ref, hp_ref, dis_ref, b_ref, batch_ref, wp_ref, bp_ref,
             pol_ref, sums_ref, cnts_ref):
        r = pl.program_id(0)

        @pl.when(r == 0)
        def _():
            sums_ref[...] = jnp.zeros((g, f), jnp.float32)
            cnts_ref[...] = jnp.zeros((g, f), jnp.float32)
            pol_ref[...] = jnp.zeros((g, nout), jnp.float32)

        dis = dis_ref[...]
        m = dis * (scat_ref[...] + hp_ref[...])
        h3 = jnp.maximum(m + b_ref[...], 0.0)
        bv = batch_ref[0, :]
        oh = (bv[:, None] == lax.broadcasted_iota(jnp.int32, (RB, g), 1))
        ohf = oh.astype(jnp.float32)
        sums_ref[...] += lax.dot_general(
            ohf, h3, (((0,), (0,)), ((), ())),
            preferred_element_type=jnp.float32,
        )
        cnts_ref[...] += lax.dot_general(
            ohf, jnp.ones((RB, f), jnp.float32), (((0,), (0,)), ((), ())),
            preferred_element_type=jnp.float32,
        )

        @pl.when(r == ng - 1)
        def _():
            pooled = sums_ref[...] / jnp.maximum(cnts_ref[...], 1.0)
            pol_ref[...] = (
                jnp.dot(pooled, wp_ref[...], preferred_element_type=jnp.float32)
                + bp_ref[...]
            )

    return pl.pallas_call(
        body,
        grid=(ng,),
        in_specs=[
            pl.BlockSpec((RB, f), lambda r: (r, 0)),
            pl.BlockSpec((RB, f), lambda r: (r, 0)),
            pl.BlockSpec((RB, 1), lambda r: (r, 0)),
            pl.BlockSpec((1, f), lambda r: (0, 0)),
            pl.BlockSpec((1, RB), lambda r: (0, r)),
            pl.BlockSpec((f, nout), lambda r: (0, 0)),
            pl.BlockSpec((1, nout), lambda r: (0, 0)),
        ],
        out_specs=pl.BlockSpec((g, nout), lambda r: (0, 0)),
        out_shape=jax.ShapeDtypeStruct((g, nout), jnp.float32),
        scratch_shapes=[
            pltpu.VMEM((g, f), jnp.float32),
            pltpu.VMEM((g, f), jnp.float32),
        ],
    )


def kernel(x, edge_index, edge_weight, batch, W1, b1, W2, b2, W3, b3, Wp, bp):
    n, fin = x.shape
    e = edge_weight.shape[0]
    f1 = W1.shape[1]
    f2 = W2.shape[1]
    f3 = W3.shape[1]
    g = 64
    nout = Wp.shape[1]

    n2 = ((n + NS * RB - 1) // (NS * RB)) * (NS * RB)
    estep = NC * NS * CH
    epad = ((e + estep - 1) // estep) * estep

    nbuk = None
    for kk in range(2, 129, 2):
        if n2 % kk == 0 and (n2 // kk) % 512 == 0 and n2 // kk <= 3600:
            nbuk = kk
            break
    assert nbuk is not None

    xp = jnp.pad(x, ((0, n2 - n), (0, 0)))
    s = jnp.pad(edge_index[0], (0, epad - e)).reshape(epad // CW, CW)
    d = jnp.pad(edge_index[1], (0, epad - e)).reshape(epad // CW, CW)
    w = jnp.pad(edge_weight, (0, epad - e)).reshape(epad // CW, CW)
    batchp = jnp.pad(batch, (0, n2 - n), constant_values=g).reshape(1, n2)

    degp = _make_deg(n2, epad)(d, w)
    dis, hp1 = _tc_prep(n2, f1)(degp.reshape(NC, n2), xp, W1)

    sB, dB, wB, cnts = _make_bin(n2, epad, nbuk)(s, d, w)

    scat1 = _make_agg(f1, n2, epad, nbuk)(hp1, sB, dB, wB, cnts)
    hp2 = _tc_layer(n2, f1, f2)(scat1, hp1, dis, b1.reshape(1, f1), W2)

    scat2 = _make_agg(f2, n2, epad, nbuk)(hp2, sB, dB, wB, cnts)
    hp3 = _tc_layer(n2, f2, f3)(scat2, hp2, dis, b2.reshape(1, f2), W3)

    scat3 = _make_agg(f3, n2, epad, nbuk)(hp3, sB, dB, wB, cnts)

    pol = _tc_pool(n2, f3, g, nout)(
        scat3, hp3, dis, b3.reshape(1, f3), batchp, Wp, bp.reshape(1, nout)
    )
    return pol


# speedup vs baseline: 7.7399x; 1.3985x over previous
import functools

import jax
import jax.numpy as jnp
from jax import lax
from jax.experimental import pallas as pl
from jax.experimental.pallas import tpu as pltpu
from jax.experimental.pallas import tpu_sc as plsc

NC = 2
NS = 16
RB = 512
CW = 128
CH = 512
CHW = 256

_SC_PARAMS = pltpu.CompilerParams(
    needs_layout_passes=False, use_tc_tiling_on_sc=False
)


def _sc_mesh():
    return plsc.VectorSubcoreMesh(
        core_axis_name="c", subcore_axis_name="s", num_cores=NC, num_subcores=NS
    )


def _make_deg(n2, epad):
    epc = epad // (NC * NS)
    nchunks = epc // CH
    nsl = n2 // NS

    @functools.partial(
        pl.kernel,
        out_type=jax.ShapeDtypeStruct((NC * n2,), jnp.float32),
        mesh=_sc_mesh(),
        scratch_types=[
            pltpu.VMEM_SHARED((n2,), jnp.float32),
            pltpu.VMEM((CH // CW, CW), jnp.int32),
            pltpu.VMEM((CH // CW, CW), jnp.float32),
            pltpu.VMEM((nsl,), jnp.float32),
        ],
        compiler_params=_SC_PARAMS,
    )
    def k(d_idx, w, out, acc, dbuf, wbuf, zbuf):
        c = lax.axis_index("c")
        sid = lax.axis_index("s")
        zv = jnp.zeros((16,), jnp.float32)

        def zfill(i, _):
            zbuf[pl.ds(i * 16, 16)] = zv
            return 0

        lax.fori_loop(0, nsl // 16, zfill, 0)
        pltpu.sync_copy(zbuf, acc.at[pl.ds(sid * nsl, nsl)])
        plsc.subcore_barrier()

        ebase = (c * NS + sid) * epc

        def chunk(ci, _):
            base = ebase + ci * CH
            pltpu.sync_copy(d_idx.at[pl.ds(base // CW, CH // CW)], dbuf)
            pltpu.sync_copy(w.at[pl.ds(base // CW, CH // CW)], wbuf)
            for j in range(CH // CW):
                pltpu.sync_copy(wbuf.at[j], acc.at[dbuf.at[j]], add=True)
            return 0

        lax.fori_loop(0, nchunks, chunk, 0)
        plsc.subcore_barrier()
        lo = sid * nsl
        pltpu.sync_copy(acc.at[pl.ds(lo, nsl)], out.at[pl.ds(c * n2 + lo, nsl)])

    return k


def _make_bin(n2, epad, nbuk):
    br = n2 // nbuk
    eps32 = epad // (NC * NS)
    rr = eps32 // CW
    orows = epad // CW

    @functools.partial(
        pl.kernel,
        out_type=[
            jax.ShapeDtypeStruct((nbuk * orows, CW), jnp.int32),
            jax.ShapeDtypeStruct((nbuk * orows, CW), jnp.float32),
            jax.ShapeDtypeStruct((nbuk * NC * NS * 16,), jnp.int32),
        ],
        mesh=_sc_mesh(),
        scratch_types=[
            pltpu.VMEM((4, CW), jnp.int32),
            pltpu.VMEM((4, CW), jnp.int32),
            pltpu.VMEM((4, CW), jnp.float32),
            pltpu.VMEM((nbuk, 288), jnp.int32),
            pltpu.VMEM((nbuk, 288), jnp.float32),
            pltpu.VMEM((16,), jnp.int32),
            pltpu.VMEM((nbuk, 16), jnp.int32),
            pltpu.VMEM((nbuk, 16), jnp.int32),
        ],
        compiler_params=_SC_PARAMS,
    )
    def k(s_in, d_in, w_in, pB, wB, cnts, sI, dI, wI, pSt, wSt, cbuf,
          fpbuf, dnbuf):
        c = lax.axis_index("c")
        sid = lax.axis_index("s")
        wid = c * NS + sid
        inrow = wid * rr
        iota = lax.iota(jnp.int32, 16)
        nullbase = (wid % 16) * 128

        def obase(b):
            return b * orows + wid * rr

        def flush(b, fp, done):
            def do_flush(_):
                row = obase(b) + done
                pltpu.sync_copy(pSt.at[b, pl.ds(0, CW)], pB.at[row])
                pltpu.sync_copy(wSt.at[b, pl.ds(0, CW)], wB.at[row])

                def mv(t, _):
                    src = pl.ds(CW + t * 16, 16)
                    dst = pl.ds(t * 16, 16)
                    pSt[b, dst] = pSt[b, src]
                    wSt[b, dst] = wSt[b, src]
                    return 0

                lax.fori_loop(0, CW // 16, mv, 0)
                return fp - CW, done + 1

            return lax.cond(fp >= CW, do_flush, lambda _: (fp, done), 0)

        def row_body(ri, carry):
            fps, dones = carry
            j = ri % 4

            @pl.when(j == 0)
            def _():
                pltpu.sync_copy(s_in.at[pl.ds(inrow + ri, 4)], sI)
                pltpu.sync_copy(d_in.at[pl.ds(inrow + ri, 4)], dI)
                pltpu.sync_copy(w_in.at[pl.ds(inrow + ri, 4)], wI)

            def vreg_body(kk, car):
                fps = car
                sl = pl.ds(kk * 16, 16)
                sv = sI[j, sl]
                dv = dI[j, sl]
                wv = wI[j, sl]
                bid = dv // br
                dloc = dv - bid * br
                pv = sv | (dloc << 17)
                out_fps = []
                for b in range(nbuk):
                    m = bid == b
                    cl = jnp.sum(m.astype(jnp.int32))
                    fp = fps[b]
                    plsc.store_compressed(pSt.at[b, pl.ds(fp, 16)], pv, mask=m)
                    plsc.store_compressed(wSt.at[b, pl.ds(fp, 16)], wv, mask=m)
                    out_fps.append(fp + cl)
                return tuple(out_fps)

            fps = lax.fori_loop(0, CW // 16, vreg_body, fps)
            new_fps, new_dones = [], []
            for b in range(nbuk):
                fp, dn = flush(b, fps[b], dones[b])
                new_fps.append(fp)
                new_dones.append(dn)
            return tuple(new_fps), tuple(new_dones)

        z = tuple(jnp.int32(0) for _ in range(nbuk))
        fps, dones = lax.fori_loop(0, rr, row_body, (z, z))
        for b in range(nbuk):
            fpbuf[b] = jnp.broadcast_to(fps[b], (16,)).astype(jnp.int32)
            dnbuf[b] = jnp.broadcast_to(dones[b], (16,)).astype(jnp.int32)

        def finish(b, fp, done):
            def pad_row(t, _):
                nulls = nullbase + t * 16 + iota
                pos = pl.ds(fp + t * 16, 16)
                pSt[b, pos] = nulls | (nulls << 17)
                wSt[b, pos] = jnp.zeros((16,), jnp.float32)
                return 0

            lax.fori_loop(0, CW // 16, pad_row, 0)
            fp2, done2 = lax.cond(
                fp > 0,
                lambda _: flush(b, fp + CW, done),
                lambda _: (fp, done),
                0,
            )

            def null_row(t, _):
                nulls = nullbase + t * 16 + iota
                pos = pl.ds(t * 16, 16)
                pSt[b, pos] = nulls | (nulls << 17)
                wSt[b, pos] = jnp.zeros((16,), jnp.float32)
                return 0

            lax.fori_loop(0, CW // 16, null_row, 0)

            def pad2(t, d2):
                def do(_):
                    row = obase(b) + d2
                    pltpu.sync_copy(pSt.at[b, pl.ds(0, CW)], pB.at[row])
                    pltpu.sync_copy(wSt.at[b, pl.ds(0, CW)], wB.at[row])
                    return d2 + 1

                return lax.cond(d2 % (CHW // CW) != 0, do, lambda _: d2, 0)

            done3 = lax.fori_loop(0, (CHW // CW) - 1, pad2, done2)
            cbuf[...] = jnp.broadcast_to(
                done3 // (CHW // CW), (16,)
            ).astype(jnp.int32)
            pltpu.sync_copy(
                cbuf, cnts.at[pl.ds((b * NC * NS + wid) * 16, 16)]
            )

        def finish_b(b, _):
            fp = jnp.max(fpbuf[b])
            done = jnp.max(dnbuf[b])
            finish(b, fp, done)
            return 0

        lax.fori_loop(0, nbuk, finish_b, 0)

    return k


def _make_agg(f, n2, epad, nbuk):
    br = n2 // nbuk
    eps32 = epad // (NC * NS)
    rr = eps32 // CW
    orows = epad // CW
    nsl = br // NS
    zr = 32
    nw = CHW // CW

    @functools.partial(
        pl.kernel,
        out_type=jax.ShapeDtypeStruct((n2, f), jnp.float32),
        mesh=_sc_mesh(),
        scratch_types=[
            pltpu.VMEM_SHARED((br, f), jnp.float32),
            pltpu.VMEM((2, nw, CW), jnp.int32),
            pltpu.VMEM((2, nw, CW), jnp.int32),
            pltpu.VMEM((2, nw, CW), jnp.int32),
            pltpu.VMEM((2, nw, CW), jnp.float32),
            pltpu.VMEM((2, CHW, f), jnp.float32),
            pltpu.VMEM((zr, f), jnp.float32),
            pltpu.VMEM((16,), jnp.int32),
            pltpu.SemaphoreType.DMA((2,)),
            pltpu.SemaphoreType.DMA((2,)),
            pltpu.SemaphoreType.DMA((2,)),
        ],
        compiler_params=_SC_PARAMS,
    )
    def k(hp, pB, wB, cnts, out, acc, pbuf, sbuf, dbuf, wbuf, rows, zrows,
          cbuf, sem_e, sem_g, sem_s):
        c = lax.axis_index("c")
        sid = lax.axis_index("s")
        zv = jnp.zeros((16,), jnp.float32)

        def zfill(i, _):
            for q in range(f // 16):
                zrows[i, pl.ds(q * 16, 16)] = zv
            return 0

        lax.fori_loop(0, zr, zfill, 0)

        r0 = sid * 2
        r1 = sid * 2 + 1
        dnums = lax.GatherDimensionNumbers(
            offset_dims=(), collapsed_slice_dims=(0,), start_index_map=(0,)
        )

        def do_bucket(bi, _):
            b = c + NC * bi
            for t in range(nsl // zr):
                pltpu.sync_copy(zrows, acc.at[pl.ds(sid * nsl + t * zr, zr)])
            plsc.subcore_barrier()

            pltpu.sync_copy(cnts.at[pl.ds((b * NC * NS + r0) * 16, 16)], cbuf)
            nch0 = jnp.max(cbuf[...])
            pltpu.sync_copy(cnts.at[pl.ds((b * NC * NS + r1) * 16, 16)], cbuf)
            nch1 = jnp.max(cbuf[...])
            n = nch0 + nch1
            base0 = b * orows + r0 * rr
            base1 = b * orows + r1 * rr

            def rowof(t):
                return jnp.where(
                    t < nch0, base0 + t * nw, base1 + (t - nch0) * nw
                )

            def fire_e(t, p):
                r = rowof(t)
                pltpu.async_copy(pB.at[pl.ds(r, nw)], pbuf.at[p], sem_e.at[p])
                pltpu.async_copy(wB.at[pl.ds(r, nw)], wbuf.at[p], sem_e.at[p])

            def wait_e(p):
                pltpu.make_async_copy(
                    pB.at[pl.ds(0, nw)], pbuf.at[p], sem_e.at[p]
                ).wait()
                pltpu.make_async_copy(
                    wB.at[pl.ds(0, nw)], wbuf.at[p], sem_e.at[p]
                ).wait()

            def unpack(p):
                def up(t, _):
                    j = t // 8
                    sl = pl.ds((t % 8) * 16, 16)
                    pv = pbuf[p, j, sl]
                    sbuf[p, j, sl] = pv & 131071
                    dbuf[p, j, sl] = pv >> 17
                    return 0

                lax.fori_loop(0, nw * (CW // 16), up, 0)

            def fire_g(p):
                for j in range(nw):
                    pltpu.async_copy(
                        hp.at[sbuf.at[p, j]],
                        rows.at[p, pl.ds(j * CW, CW)],
                        sem_g.at[p],
                    )

            def wait_g(p):
                for j in range(nw):
                    pltpu.make_async_copy(
                        hp.at[sbuf.at[p, j]],
                        rows.at[p, pl.ds(j * CW, CW)],
                        sem_g.at[p],
                    ).wait()

            def fire_s(p):
                for j in range(nw):
                    pltpu.async_copy(
                        rows.at[p, pl.ds(j * CW, CW)],
                        acc.at[dbuf.at[p, j]],
                        sem_s.at[p],
                        add=True,
                    )

            def wait_s(p):
                for j in range(nw):
                    pltpu.make_async_copy(
                        rows.at[p, pl.ds(j * CW, CW)],
                        acc.at[dbuf.at[p, j]],
                        sem_s.at[p],
                    ).wait()

            def scale(p):
                def sc_body(j, _):
                    wv = wbuf[p, j // 8, pl.ds((j % 8) * 16, 16)]
                    for kk in range(16):
                        i = j * 16 + kk
                        idx = jnp.full((16, 1), kk, jnp.int32)
                        bc = lax.gather(
                            wv, idx, dnums, (1,),
                            mode=lax.GatherScatterMode.PROMISE_IN_BOUNDS,
                        )
                        for q in range(f // 16):
                            sl = pl.ds(q * 16, 16)
                            rows[p, i, sl] = rows[p, i, sl] * bc
                    return 0

                lax.fori_loop(0, CHW // 16, sc_body, 0)

            @pl.when(n > 0)
            def _():
                fire_e(0, 0)

            def pipe(m, _):
                a = 2 * m
                bch = 2 * m + 1
                prev_odd = (a - 1 >= 0) & (a - 1 < n)

                @pl.when(a < n)
                def _():
                    wait_e(0)
                    unpack(0)
                    fire_g(0)

                @pl.when(prev_odd)
                def _():
                    wait_g(1)
                    scale(1)
                    fire_s(1)

                @pl.when(a < n)
                def _():
                    wait_g(0)
                    scale(0)

                @pl.when(prev_odd)
                def _():
                    wait_s(1)

                @pl.when(bch < n)
                def _():
                    fire_e(bch, 1)

                @pl.when(a < n)
                def _():
                    fire_s(0)

                @pl.when(bch < n)
                def _():
                    wait_e(1)
                    unpack(1)
                    fire_g(1)

                @pl.when(a < n)
                def _():
                    wait_s(0)

                @pl.when(a + 2 < n)
                def _():
                    fire_e(a + 2, 0)

                return 0

            lax.fori_loop(0, n // 2 + 1, pipe, 0)
            plsc.subcore_barrier()
            lo = sid * nsl
            pltpu.sync_copy(
                acc.at[pl.ds(lo, nsl)], out.at[pl.ds(b * br + lo, nsl)]
            )
            plsc.subcore_barrier()
            return 0

        lax.fori_loop(0, nbuk // NC, do_bucket, 0)

    return k


def _tc_prep(n2, f1):
    ng = n2 // RB

    def body(deg_ref, x_ref, w1_ref, dis_ref, hp_ref):
        deg = deg_ref[0, :] + deg_ref[1, :] + 1.0
        dis = lax.rsqrt(jnp.maximum(deg, 1e-12))
        h = jnp.dot(x_ref[...], w1_ref[...], preferred_element_type=jnp.float32)
        dis_ref[...] = dis[:, None]
        hp_ref[...] = h * dis[:, None]

    return pl.pallas_call(
        body,
        grid=(ng,),
        in_specs=[
            pl.BlockSpec((2, RB), lambda r: (0, r)),
            pl.BlockSpec((RB, 32), lambda r: (r, 0)),
            pl.BlockSpec((32, f1), lambda r: (0, 0)),
        ],
        out_specs=[
            pl.BlockSpec((RB, 1), lambda r: (r, 0)),
            pl.BlockSpec((RB, f1), lambda r: (r, 0)),
        ],
        out_shape=[
            jax.ShapeDtypeStruct((n2, 1), jnp.float32),
            jax.ShapeDtypeStruct((n2, f1), jnp.float32),
        ],
    )


def _tc_layer(n2, f, f2):
    ng = n2 // RB

    def body(scat_ref, hp_ref, dis_ref, b_ref, w_ref, hp2_ref):
        dis = dis_ref[...]
        m = dis * (scat_ref[...] + hp_ref[...])
        act = jnp.maximum(m + b_ref[...], 0.0)
        h2 = jnp.dot(act, w_ref[...], preferred_element_type=jnp.float32)
        hp2_ref[...] = h2 * dis

    return pl.pallas_call(
        body,
        grid=(ng,),
        in_specs=[
            pl.BlockSpec((RB, f), lambda r: (r, 0)),
            pl.BlockSpec((RB, f), lambda r: (r, 0)),
            pl.BlockSpec((RB, 1), lambda r: (r, 0)),
            pl.BlockSpec((1, f), lambda r: (0, 0)),
            pl.BlockSpec((f, f2), lambda r: (0, 0)),
        ],
        out_specs=pl.BlockSpec((RB, f2), lambda r: (r, 0)),
        out_shape=jax.ShapeDtypeStruct((n2, f2), jnp.float32),
    )


def _tc_pool(n2, f, g, nout):
    ng = n2 // RB

    def body(scat_ref, hp_ref, dis_ref, b_ref, batch_ref, wp_ref, bp_ref,
             pol_ref, sums_ref, cnts_ref):
        r = pl.program_id(0)

        @pl.when(r == 0)
        def _():
            sums_ref[...] = jnp.zeros((g, f), jnp.float32)
            cnts_ref[...] = jnp.zeros((g, f), jnp.float32)
            pol_ref[...] = jnp.zeros((g, nout), jnp.float32)

        dis = dis_ref[...]
        m = dis * (scat_ref[...] + hp_ref[...])
        h3 = jnp.maximum(m + b_ref[...], 0.0)
        bv = batch_ref[0, :]
        oh = (bv[:, None] == lax.broadcasted_iota(jnp.int32, (RB, g), 1))
        ohf = oh.astype(jnp.float32)
        sums_ref[...] += lax.dot_general(
            ohf, h3, (((0,), (0,)), ((), ())),
            preferred_element_type=jnp.float32,
        )
        cnts_ref[...] += lax.dot_general(
            ohf, jnp.ones((RB, f), jnp.float32), (((0,), (0,)), ((), ())),
            preferred_element_type=jnp.float32,
        )

        @pl.when(r == ng - 1)
        def _():
            pooled = sums_ref[...] / jnp.maximum(cnts_ref[...], 1.0)
            pol_ref[...] = (
                jnp.dot(pooled, wp_ref[...], preferred_element_type=jnp.float32)
                + bp_ref[...]
            )

    return pl.pallas_call(
        body,
        grid=(ng,),
        in_specs=[
            pl.BlockSpec((RB, f), lambda r: (r, 0)),
            pl.BlockSpec((RB, f), lambda r: (r, 0)),
            pl.BlockSpec((RB, 1), lambda r: (r, 0)),
            pl.BlockSpec((1, f), lambda r: (0, 0)),
            pl.BlockSpec((1, RB), lambda r: (0, r)),
            pl.BlockSpec((f, nout), lambda r: (0, 0)),
            pl.BlockSpec((1, nout), lambda r: (0, 0)),
        ],
        out_specs=pl.BlockSpec((g, nout), lambda r: (0, 0)),
        out_shape=jax.ShapeDtypeStruct((g, nout), jnp.float32),
        scratch_shapes=[
            pltpu.VMEM((g, f), jnp.float32),
            pltpu.VMEM((g, f), jnp.float32),
        ],
    )


def kernel(x, edge_index, edge_weight, batch, W1, b1, W2, b2, W3, b3, Wp, bp):
    n, fin = x.shape
    e = edge_weight.shape[0]
    f1 = W1.shape[1]
    f2 = W2.shape[1]
    f3 = W3.shape[1]
    g = 64
    nout = Wp.shape[1]

    n2 = ((n + NS * RB - 1) // (NS * RB)) * (NS * RB)
    estep = NC * NS * CH
    epad = ((e + estep - 1) // estep) * estep

    nbuk = None
    for kk in range(2, 129, 2):
        if n2 % kk == 0 and (n2 // kk) % 512 == 0 and n2 // kk <= 3600:
            nbuk = kk
            break
    assert nbuk is not None

    xp = jnp.pad(x, ((0, n2 - n), (0, 0)))
    s = jnp.pad(edge_index[0], (0, epad - e)).reshape(epad // CW, CW)
    d = jnp.pad(edge_index[1], (0, epad - e)).reshape(epad // CW, CW)
    w = jnp.pad(edge_weight, (0, epad - e)).reshape(epad // CW, CW)
    batchp = jnp.pad(batch, (0, n2 - n), constant_values=g).reshape(1, n2)

    degp = _make_deg(n2, epad)(d, w)
    dis, hp1 = _tc_prep(n2, f1)(degp.reshape(NC, n2), xp, W1)

    pB, wB, cnts = _make_bin(n2, epad, nbuk)(s, d, w)

    scat1 = _make_agg(f1, n2, epad, nbuk)(hp1, pB, wB, cnts)
    hp2 = _tc_layer(n2, f1, f2)(scat1, hp1, dis, b1.reshape(1, f1), W2)

    scat2 = _make_agg(f2, n2, epad, nbuk)(hp2, pB, wB, cnts)
    hp3 = _tc_layer(n2, f2, f3)(scat2, hp2, dis, b2.reshape(1, f2), W3)

    scat3 = _make_agg(f3, n2, epad, nbuk)(hp3, pB, wB, cnts)

    pol = _tc_pool(n2, f3, g, nout)(
        scat3, hp3, dis, b3.reshape(1, f3), batchp, Wp, bp.reshape(1, nout)
    )
    return pol
